# baseline (speedup 1.0000x reference)
import functools

import jax
import jax.numpy as jnp
from jax import lax
from jax.experimental import pallas as pl
from jax.experimental.pallas import tpu as pltpu

B, L, K, V, NUM_OUT, TOPK = 128, 64, 1024, 30522, 18, 32
VT = 512
NV = (V + VT - 1) // VT
J = B * L
CH = 128
JP = 16384
NCH = JP // CH


def _body(bounds_ref, w_ref, v_ref, pos_ref, nz_ref, b_ref, wout_ref,
          bout_ref, out_ref, acc_ref, c_ref, s_ref, msk_ref):
    t = pl.program_id(0)

    @pl.when(t == 0)
    def _init():
        acc_ref[...] = jnp.zeros_like(acc_ref)
        p = pos_ref[...]
        c_ref[...] = jnp.cos(p)
        s_ref[...] = jnp.sin(p)

    w = w_ref[...]
    cols = lax.broadcasted_iota(jnp.int32, (K, 2 * VT), 1)
    w = jnp.where(cols < (2 * V - t * (2 * VT)), w, 0.0)
    w_hi = w.astype(jnp.bfloat16)
    w_lo = (w - w_hi.astype(jnp.float32)).astype(jnp.bfloat16)

    c0 = bounds_ref[t] // CH
    c1 = bounds_ref[t + 1] // CH
    half_iota = lax.broadcasted_iota(jnp.int32, (2 * VT, CH), 0) // 2
    even_row = lax.broadcasted_iota(jnp.int32, (2 * VT, CH), 0) % 2 == 0
    biota = lax.broadcasted_iota(jnp.int32, (B, CH), 0)

    def one_chunk(c):
        cc = jnp.minimum(c, NCH - 1)
        live = (c < c1).astype(jnp.float32)
        vj = v_ref[pl.ds(cc, 1), :]
        dv = vj - t * VT
        inr = (dv >= 0) & (dv < VT)
        p = half_iota == dv
        oh_re = (p & even_row).astype(jnp.bfloat16)
        oh_im = (p & (~even_row)).astype(jnp.bfloat16)
        dn = (((1,), (0,)), ((), ()))

        def gmm(lhs, rhs):
            return lax.dot_general(lhs, rhs, dn,
                                   preferred_element_type=jnp.float32)

        g_re = gmm(w_hi, oh_re) + gmm(w_lo, oh_re)
        g_im = gmm(w_hi, oh_im) + gmm(w_lo, oh_im)
        cj = c_ref[pl.ds(cc, 1), :]
        sj = s_ref[pl.ds(cc, 1), :]
        fac = nz_ref[pl.ds(cc, 1), :] * inr.astype(jnp.float32) * live
        wabs = jnp.sqrt(g_re * g_re + g_im * g_im)
        re = g_re * cj + g_im * sj
        im = g_im * cj - g_re * sj
        tt = jnp.float32(1e-10)
        ar = jnp.abs(re)
        ay = jnp.maximum(jnp.abs(im), tt)
        ax = jnp.maximum(ar, tt)
        neg = (re < 0) & (ar >= tt)
        mn = jnp.minimum(ax, ay)
        mx = jnp.maximum(ax, ay)
        r = mn / mx
        t2 = r * r
        pp = jnp.float32(-0.0117212)
        pp = pp * t2 + jnp.float32(0.05265332)
        pp = pp * t2 + jnp.float32(-0.11643287)
        pp = pp * t2 + jnp.float32(0.19354346)
        pp = pp * t2 + jnp.float32(-0.33262347)
        pp = pp * t2 + jnp.float32(0.99997726)
        a = r * pp
        hpi = jnp.float32(jnp.pi / 2)
        a = jnp.where(ay > ax, hpi - a, a)
        phi = jnp.where(neg, jnp.float32(jnp.pi) - a, a)
        f = (wabs + phi) * fac
        bj = b_ref[pl.ds(cc, 1), :]
        e = (biota == bj).astype(jnp.bfloat16)
        f_hi = f.astype(jnp.bfloat16)
        f_lo = (f - f_hi.astype(jnp.float32)).astype(jnp.bfloat16)
        dn2 = (((1,), (1,)), ((), ()))
        return (lax.dot_general(e, f_hi, dn2,
                                preferred_element_type=jnp.float32) +
                lax.dot_general(e, f_lo, dn2,
                                preferred_element_type=jnp.float32))

    def chunk_body(u, carry):
        c = c0 + 2 * u
        acc_ref[...] += one_chunk(c) + one_chunk(c + 1)
        return carry

    lax.fori_loop(0, (c1 - c0 + 1) // 2, chunk_body, 0)

    @pl.when(t == NV - 1)
    def _finish():
        a = acc_ref[...]
        kiota = lax.broadcasted_iota(jnp.int32, (B, K), 1)
        msk_ref[...] = jnp.zeros_like(msk_ref)

        def round_body(r, carry):
            am = jnp.where(msk_ref[...] > 0, -1.0, a)
            m = jnp.max(am, axis=1, keepdims=True)
            eq = am == m
            idx = jnp.min(jnp.where(eq, kiota, K + K), axis=1,
                          keepdims=True)
            msk_ref[...] = jnp.where(kiota == idx, 1.0, msk_ref[...])
            return carry

        lax.fori_loop(0, TOPK, round_body, 0)
        feats = msk_ref[...]
        logits = lax.dot_general(feats, wout_ref[...],
                                 (((1,), (1,)), ((), ())),
                                 preferred_element_type=jnp.float32,
                                 precision=lax.Precision.HIGHEST)
        out_ref[...] = logits + bout_ref[...]


@jax.jit
def kernel(obs, W_r, W_out, b_out):
    ids = jnp.squeeze(obs, axis=1)
    nz = ids != 0
    position_ids = jnp.cumsum(nz, axis=-1).astype(jnp.float32)
    slen = nz.sum(-1).astype(jnp.float32)
    any_nz = nz.sum() > 0
    pos = jnp.where(any_nz,
                    jnp.pi * position_ids / slen[:, None],
                    jnp.pi * position_ids / float(L))

    v = ids.reshape(-1)
    perm = jnp.argsort(v)
    v_s = v[perm].astype(jnp.int32)
    pos_s = pos.reshape(-1)[perm]
    nz_s = nz.reshape(-1)[perm].astype(jnp.float32)
    b_s = (perm // L).astype(jnp.int32)
    edges = jnp.arange(NV + 1, dtype=jnp.int32) * VT
    raw_bounds = jnp.searchsorted(v_s, edges).astype(jnp.int32)
    cnt = raw_bounds[1:] - raw_bounds[:-1]
    padded = ((cnt + CH - 1) // CH) * CH
    offs = jnp.concatenate([jnp.zeros((1,), jnp.int32),
                            jnp.cumsum(padded).astype(jnp.int32)])
    tile_of = v_s // VT
    dest = offs[tile_of] + (jnp.arange(J, dtype=jnp.int32)
                            - raw_bounds[tile_of])
    v_p = jnp.zeros((JP,), jnp.int32).at[dest].set(v_s)
    pos_p = jnp.zeros((JP,), jnp.float32).at[dest].set(pos_s)
    nz_p = jnp.zeros((JP,), jnp.float32).at[dest].set(nz_s)
    b_p = jnp.zeros((JP,), jnp.int32).at[dest].set(b_s)
    bounds = offs

    w2 = W_r.reshape(K, 2 * V)

    grid_spec = pltpu.PrefetchScalarGridSpec(
        num_scalar_prefetch=1,
        grid=(NV,),
        in_specs=[
            pl.BlockSpec((K, 2 * VT), lambda t, b: (0, t)),
            pl.BlockSpec((NCH, CH), lambda t, b: (0, 0)),
            pl.BlockSpec((NCH, CH), lambda t, b: (0, 0)),
            pl.BlockSpec((NCH, CH), lambda t, b: (0, 0)),
            pl.BlockSpec((NCH, CH), lambda t, b: (0, 0)),
            pl.BlockSpec((NUM_OUT, K), lambda t, b: (0, 0)),
            pl.BlockSpec((1, NUM_OUT), lambda t, b: (0, 0)),
        ],
        out_specs=pl.BlockSpec((B, NUM_OUT), lambda t, b: (0, 0)),
        scratch_shapes=[
            pltpu.VMEM((B, K), jnp.float32),
            pltpu.VMEM((NCH, CH), jnp.float32),
            pltpu.VMEM((NCH, CH), jnp.float32),
            pltpu.VMEM((B, K), jnp.float32),
        ],
    )
    out = pl.pallas_call(
        _body,
        grid_spec=grid_spec,
        out_shape=jax.ShapeDtypeStruct((B, NUM_OUT), jnp.float32),
    )(bounds, w2, v_p.reshape(NCH, CH), pos_p.reshape(NCH, CH),
      nz_p.reshape(NCH, CH), b_p.reshape(NCH, CH), W_out,
      b_out.reshape(1, NUM_OUT))
    return out


# speedup vs baseline: 1.1625x; 1.1625x over previous
import functools

import jax
import jax.numpy as jnp
from jax import lax
from jax.experimental import pallas as pl
from jax.experimental.pallas import tpu as pltpu

B, L, K, V, NUM_OUT, TOPK = 128, 64, 1024, 30522, 18, 32
VT = 512
NV = (V + VT - 1) // VT
J = B * L
CH = 128
NCH = J // CH


def _body(bounds_ref, w_ref, v_ref, pos_ref, nz_ref, b_ref, wout_ref,
          bout_ref, out_ref, acc_ref, c_ref, s_ref, msk_ref):
    t = pl.program_id(0)

    @pl.when(t == 0)
    def _init():
        acc_ref[...] = jnp.zeros_like(acc_ref)
        p = pos_ref[...]
        c_ref[...] = jnp.cos(p)
        s_ref[...] = jnp.sin(p)

    w = w_ref[...]
    cols = lax.broadcasted_iota(jnp.int32, (K, 2 * VT), 1)
    w = jnp.where(cols < (2 * V - t * (2 * VT)), w, 0.0)
    w_hi = w.astype(jnp.bfloat16)
    w_lo = (w - w_hi.astype(jnp.float32)).astype(jnp.bfloat16)

    s0 = bounds_ref[t]
    e0 = bounds_ref[t + 1]
    c0 = s0 // CH
    c1 = (e0 + CH - 1) // CH
    half_iota = lax.broadcasted_iota(jnp.int32, (2 * VT, CH), 0) // 2
    even_row = lax.broadcasted_iota(jnp.int32, (2 * VT, CH), 0) % 2 == 0
    biota = lax.broadcasted_iota(jnp.int32, (B, CH), 0)

    def one_chunk(c):
        cc = jnp.minimum(c, NCH - 1)
        live = (c < c1).astype(jnp.float32)
        vj = v_ref[pl.ds(cc, 1), :]
        dv = vj - t * VT
        inr = (dv >= 0) & (dv < VT)
        p = half_iota == dv
        oh_re = (p & even_row).astype(jnp.bfloat16)
        oh_im = (p & (~even_row)).astype(jnp.bfloat16)
        dn = (((1,), (0,)), ((), ()))

        def gmm(lhs, rhs):
            return lax.dot_general(lhs, rhs, dn,
                                   preferred_element_type=jnp.float32)

        g_re = gmm(w_hi, oh_re) + gmm(w_lo, oh_re)
        g_im = gmm(w_hi, oh_im) + gmm(w_lo, oh_im)
        cj = c_ref[pl.ds(cc, 1), :]
        sj = s_ref[pl.ds(cc, 1), :]
        fac = nz_ref[pl.ds(cc, 1), :] * inr.astype(jnp.float32) * live
        wabs = jnp.sqrt(g_re * g_re + g_im * g_im)
        re = g_re * cj + g_im * sj
        im = g_im * cj - g_re * sj
        tt = jnp.float32(1e-10)
        ar = jnp.abs(re)
        ay = jnp.maximum(jnp.abs(im), tt)
        ax = jnp.maximum(ar, tt)
        neg = (re < 0) & (ar >= tt)
        mn = jnp.minimum(ax, ay)
        mx = jnp.maximum(ax, ay)
        r = mn / mx
        t2 = r * r
        pp = jnp.float32(-0.0117212)
        pp = pp * t2 + jnp.float32(0.05265332)
        pp = pp * t2 + jnp.float32(-0.11643287)
        pp = pp * t2 + jnp.float32(0.19354346)
        pp = pp * t2 + jnp.float32(-0.33262347)
        pp = pp * t2 + jnp.float32(0.99997726)
        a = r * pp
        hpi = jnp.float32(jnp.pi / 2)
        a = jnp.where(ay > ax, hpi - a, a)
        phi = jnp.where(neg, jnp.float32(jnp.pi) - a, a)
        f = (wabs + phi) * fac
        bj = b_ref[pl.ds(cc, 1), :]
        e = (biota == bj).astype(jnp.bfloat16)
        f_hi = f.astype(jnp.bfloat16)
        f_lo = (f - f_hi.astype(jnp.float32)).astype(jnp.bfloat16)
        dn2 = (((1,), (1,)), ((), ()))
        return (lax.dot_general(e, f_hi, dn2,
                                preferred_element_type=jnp.float32) +
                lax.dot_general(e, f_lo, dn2,
                                preferred_element_type=jnp.float32))

    def chunk_body(u, carry):
        c = c0 + 2 * u
        acc_ref[...] += one_chunk(c) + one_chunk(c + 1)
        return carry

    lax.fori_loop(0, (c1 - c0 + 1) // 2, chunk_body, 0)

    @pl.when(t == NV - 1)
    def _finish():
        a = acc_ref[...]
        kiota = lax.broadcasted_iota(jnp.int32, (B, K), 1)
        msk_ref[...] = jnp.zeros_like(msk_ref)

        def round_body(r, carry):
            am = jnp.where(msk_ref[...] > 0, -1.0, a)
            m = jnp.max(am, axis=1, keepdims=True)
            eq = am == m
            idx = jnp.min(jnp.where(eq, kiota, K + K), axis=1,
                          keepdims=True)
            msk_ref[...] = jnp.where(kiota == idx, 1.0, msk_ref[...])
            return carry

        lax.fori_loop(0, TOPK, round_body, 0)
        feats = msk_ref[...]
        logits = lax.dot_general(feats, wout_ref[...],
                                 (((1,), (1,)), ((), ())),
                                 preferred_element_type=jnp.float32,
                                 precision=lax.Precision.HIGHEST)
        out_ref[...] = logits + bout_ref[...]


@jax.jit
def kernel(obs, W_r, W_out, b_out):
    ids = jnp.squeeze(obs, axis=1)
    nz = ids != 0
    position_ids = jnp.cumsum(nz, axis=-1).astype(jnp.float32)
    slen = nz.sum(-1).astype(jnp.float32)
    any_nz = nz.sum() > 0
    pos = jnp.where(any_nz,
                    jnp.pi * position_ids / slen[:, None],
                    jnp.pi * position_ids / float(L))

    v = ids.reshape(-1)
    perm = jnp.argsort(v)
    v_s = v[perm].astype(jnp.int32)
    pos_s = pos.reshape(-1)[perm]
    nz_s = nz.reshape(-1)[perm].astype(jnp.float32)
    b_s = (perm // L).astype(jnp.int32)
    edges = jnp.arange(NV + 1, dtype=jnp.int32) * VT
    bounds = jnp.searchsorted(v_s, edges).astype(jnp.int32)

    w2 = W_r.reshape(K, 2 * V)

    grid_spec = pltpu.PrefetchScalarGridSpec(
        num_scalar_prefetch=1,
        grid=(NV,),
        in_specs=[
            pl.BlockSpec((K, 2 * VT), lambda t, b: (0, t)),
            pl.BlockSpec((NCH, CH), lambda t, b: (0, 0)),
            pl.BlockSpec((NCH, CH), lambda t, b: (0, 0)),
            pl.BlockSpec((NCH, CH), lambda t, b: (0, 0)),
            pl.BlockSpec((NCH, CH), lambda t, b: (0, 0)),
            pl.BlockSpec((NUM_OUT, K), lambda t, b: (0, 0)),
            pl.BlockSpec((1, NUM_OUT), lambda t, b: (0, 0)),
        ],
        out_specs=pl.BlockSpec((B, NUM_OUT), lambda t, b: (0, 0)),
        scratch_shapes=[
            pltpu.VMEM((B, K), jnp.float32),
            pltpu.VMEM((NCH, CH), jnp.float32),
            pltpu.VMEM((NCH, CH), jnp.float32),
            pltpu.VMEM((B, K), jnp.float32),
        ],
    )
    out = pl.pallas_call(
        _body,
        grid_spec=grid_spec,
        out_shape=jax.ShapeDtypeStruct((B, NUM_OUT), jnp.float32),
    )(bounds, w2, v_s.reshape(NCH, CH), pos_s.reshape(NCH, CH),
      nz_s.reshape(NCH, CH), b_s.reshape(NCH, CH), W_out,
      b_out.reshape(1, NUM_OUT))
    return out
